# baseline (speedup 1.0000x reference)
import functools

import jax
import jax.numpy as jnp
from jax import lax
from jax.experimental import pallas as pl
from jax.experimental.pallas import tpu as pltpu
from jax.experimental.pallas import tpu_sc as plsc

_NC = 2
_NS = 16
_L = 16


def _sc_body(cw_hbm, rw_hbm, o_hbm, tabc, tabr, buf, sem):
    d = cw_hbm.shape[1]
    h = rw_hbm.shape[0]
    w = cw_hbm.shape[0]
    bs = o_hbm.shape[0]
    hw = h * w
    nchunk = (hw + _L - 1) // _L
    wid = lax.axis_index("s") * _NC + lax.axis_index("c")
    c0 = wid * _L
    is_col = c0 < d
    tcol = jnp.where(is_col, c0, c0 - d)

    pltpu.sync_copy(cw_hbm, tabc)
    pltpu.sync_copy(rw_hbm, tabr)

    iota = jnp.arange(_L, dtype=jnp.int32)

    @pl.when(is_col)
    def _():
        for cl in range(_L):
            col_idx = jnp.full((_L,), tcol + cl, dtype=jnp.int32)

            def body(kc, _):
                k0 = jnp.minimum(kc * _L, hw - _L)
                v = plsc.load_gather(tabc, [(k0 + iota) % w, col_idx])
                buf[cl, pl.ds(k0, _L)] = v
                return 0

            lax.fori_loop(0, nchunk, body, 0)

    @pl.when(jnp.logical_not(is_col))
    def _():
        for cl in range(_L):
            col_idx = jnp.full((_L,), tcol + cl, dtype=jnp.int32)

            def body(kc, _):
                k0 = jnp.minimum(kc * _L, hw - _L)
                v = plsc.load_gather(tabr, [(k0 + iota) // w, col_idx])
                buf[cl, pl.ds(k0, _L)] = v
                return 0

            lax.fori_loop(0, nchunk, body, 0)

    for b in range(bs):
        pltpu.async_copy(buf, o_hbm.at[b, pl.ds(c0, _L)], sem)
    for b in range(bs):
        pltpu.make_async_copy(buf, o_hbm.at[b, pl.ds(c0, _L)], sem).wait()


def kernel(mask, row_weight, col_weight):
    bs, h, w = mask.shape
    d = row_weight.shape[1]
    mesh = plsc.VectorSubcoreMesh(core_axis_name="c", subcore_axis_name="s")
    sck = functools.partial(
        pl.kernel,
        out_type=jax.ShapeDtypeStruct((bs, 2 * d, h * w), jnp.float32),
        mesh=mesh,
        scratch_types=[
            pltpu.VMEM((w, d), jnp.float32),
            pltpu.VMEM((h, d), jnp.float32),
            pltpu.VMEM((_L, h * w), jnp.float32),
            pltpu.SemaphoreType.DMA,
        ],
        compiler_params=pltpu.CompilerParams(
            needs_layout_passes=False, use_tc_tiling_on_sc=False
        ),
    )(_sc_body)
    pos_flat = sck(col_weight, row_weight)
    eps = jnp.max(row_weight) * jnp.float32(0.0)
    return jnp.reshape(pos_flat, (bs, 2 * d, h, w)) + eps


# speedup vs baseline: 1.3690x; 1.3690x over previous
import functools

import jax
import jax.numpy as jnp
from jax import lax
from jax.experimental import pallas as pl
from jax.experimental.pallas import tpu as pltpu
from jax.experimental.pallas import tpu_sc as plsc

_NC = 2
_NS = 16
_L = 16


def _slab_body(cw_ref, rw_ref, o_ref):
    cwT = cw_ref[...].T
    rwT = rw_ref[...].T
    d, w = cwT.shape
    h = rwT.shape[1]
    o_ref[0:d] = jnp.broadcast_to(cwT[:, None, :], (d, h, w))
    o_ref[d:] = jnp.broadcast_to(rwT[:, :, None], (d, h, w))


def _bcast_body(slab_hbm, o_hbm, buf, sem):
    bs = o_hbm.shape[0]
    wid = lax.axis_index("s") * _NC + lax.axis_index("c")
    c0 = wid * _L
    pltpu.sync_copy(slab_hbm.at[pl.ds(c0, _L)], buf)
    for b in range(bs):
        pltpu.async_copy(buf, o_hbm.at[b, pl.ds(c0, _L)], sem)
    for b in range(bs):
        pltpu.make_async_copy(buf, o_hbm.at[b, pl.ds(c0, _L)], sem).wait()


def kernel(mask, row_weight, col_weight):
    bs, h, w = mask.shape
    d = row_weight.shape[1]

    slab = pl.pallas_call(
        _slab_body,
        in_specs=[
            pl.BlockSpec(memory_space=pltpu.VMEM),
            pl.BlockSpec(memory_space=pltpu.VMEM),
        ],
        out_specs=pl.BlockSpec(memory_space=pltpu.VMEM),
        out_shape=jax.ShapeDtypeStruct((2 * d, h, w), jnp.float32),
    )(col_weight, row_weight)

    mesh = plsc.VectorSubcoreMesh(core_axis_name="c", subcore_axis_name="s")
    bcast = functools.partial(
        pl.kernel,
        out_type=jax.ShapeDtypeStruct((bs, 2 * d, h, w), jnp.float32),
        mesh=mesh,
        scratch_types=[
            pltpu.VMEM((_L, h, w), jnp.float32),
            pltpu.SemaphoreType.DMA,
        ],
        compiler_params=pltpu.CompilerParams(use_tc_tiling_on_sc=True),
    )(_bcast_body)
    return bcast(slab)
